# baseline (speedup 1.0000x reference)
import functools

import jax
import jax.numpy as jnp
from jax import lax
from jax.experimental import pallas as pl
from jax.experimental.pallas import tpu as pltpu
from jax.experimental.pallas import tpu_sc as plsc

N_TOK = 16384
D = 128
N_BITS = 17

BUCKET_SHIFT = 10
N_BUCKETS = 1 << (N_BITS - BUCKET_SHIFT)

SC_CORES = 2
SC_SUBCORES = 16
SC_WORKERS = SC_CORES * SC_SUBCORES
ROWS_PER_WORKER = N_TOK // SC_WORKERS
GATHER_CHUNK = 128
N_CHUNKS = ROWS_PER_WORKER // GATHER_CHUNK

DENSE_BLK = 1024
CMB_BLK = 512
N_CMB = N_TOK // CMB_BLK
ST_W = 256


def _dense_body(x_ref, wkT_ref, bk_ref, wvT_ref, bv_ref, wcT_ref, bc_ref,
                w1T_ref, b1_ref, w2_ref, b2_ref, p_ref,
                vals_ref, g_ref, idx_ref, cnt_ref):
    xb = x_ref[...]
    keys = jnp.dot(xb, wkT_ref[...],
                   preferred_element_type=jnp.float32) + bk_ref[...]
    vals = jnp.dot(xb, wvT_ref[...],
                   preferred_element_type=jnp.float32) + bv_ref[...]
    ctx = jnp.dot(xb, wcT_ref[...],
                  preferred_element_type=jnp.float32) + bc_ref[...]
    h = jax.nn.relu(jnp.dot(ctx, w1T_ref[...],
                            preferred_element_type=jnp.float32) + b1_ref[...])
    glogit = jnp.sum(h * w2_ref[...], axis=1, keepdims=True) + b2_ref[...]
    g = jax.nn.sigmoid(glogit)
    s = jnp.dot(keys, p_ref[...],
                preferred_element_type=jnp.float32)
    bits = (s > 0.0).astype(jnp.int32)
    powers = jnp.left_shift(
        1, lax.broadcasted_iota(jnp.int32, (1, N_BITS), 1))
    idx = jnp.sum(bits * powers, axis=1, keepdims=True)
    vals_ref[...] = vals
    g_ref[...] = g
    idx_ref[...] = idx
    b = idx >> BUCKET_SHIFT
    iota_b = lax.broadcasted_iota(jnp.int32, (1, N_BUCKETS), 1)
    oh = jnp.where(b == iota_b, 1.0, 0.0)
    c0 = jnp.sum(oh[:CMB_BLK], axis=0, keepdims=True)
    c1 = jnp.sum(oh[CMB_BLK:], axis=0, keepdims=True)
    cnt_ref[...] = jnp.concatenate([c0, c1], axis=0).reshape(1, 2, N_BUCKETS)


def _dense_proj(x, WkT, bk, WvT, bv, WcT, bc, W1T, b1, W2r, b2, P):
    n_blk = N_TOK // DENSE_BLK
    full = lambda shape: pl.BlockSpec(shape, lambda i: (0, 0))
    return pl.pallas_call(
        _dense_body,
        grid=(n_blk,),
        in_specs=[
            pl.BlockSpec((DENSE_BLK, D), lambda i: (i, 0)),
            full(WkT.shape), full(bk.shape),
            full(WvT.shape), full(bv.shape),
            full(WcT.shape), full(bc.shape),
            full(W1T.shape), full(b1.shape),
            full(W2r.shape), full(b2.shape),
            full(P.shape),
        ],
        out_specs=[
            pl.BlockSpec((DENSE_BLK, D), lambda i: (i, 0)),
            pl.BlockSpec((DENSE_BLK, 1), lambda i: (i, 0)),
            pl.BlockSpec((DENSE_BLK, 1), lambda i: (i, 0)),
            pl.BlockSpec((1, 2, N_BUCKETS), lambda i: (i, 0, 0)),
        ],
        out_shape=[
            jax.ShapeDtypeStruct((N_TOK, D), jnp.float32),
            jax.ShapeDtypeStruct((N_TOK, 1), jnp.float32),
            jax.ShapeDtypeStruct((N_TOK, 1), jnp.int32),
            jax.ShapeDtypeStruct((n_blk, 2, N_BUCKETS), jnp.float32),
        ],
    )(x, WkT, bk, WvT, bv, WcT, bc, W1T, b1, W2r, b2, P)


def _sc_gather_kernel(table_hbm, idx_hbm, out_hbm, idx_v, rows_v, sem):
    wid = lax.axis_index("s") * SC_CORES + lax.axis_index("c")
    base = wid * ROWS_PER_WORKER
    pltpu.sync_copy(idx_hbm.at[pl.ds(wid * N_CHUNKS, N_CHUNKS)], idx_v)
    for c in range(N_CHUNKS):
        pltpu.async_copy(
            table_hbm.at[idx_v.at[c]],
            rows_v.at[pl.ds(c * GATHER_CHUNK, GATHER_CHUNK)],
            sem,
        ).wait()
    pltpu.sync_copy(rows_v, out_hbm.at[pl.ds(base, ROWS_PER_WORKER)])


def _sc_gather(table, idx_mat):
    mesh = plsc.VectorSubcoreMesh(core_axis_name="c", subcore_axis_name="s")
    kern = functools.partial(
        pl.kernel,
        mesh=mesh,
        out_type=jax.ShapeDtypeStruct((N_TOK, D), jnp.float32),
        scratch_types=[
            pltpu.VMEM((N_CHUNKS, GATHER_CHUNK), jnp.int32),
            pltpu.VMEM((ROWS_PER_WORKER, D), jnp.float32),
            pltpu.SemaphoreType.DMA,
        ],
    )(_sc_gather_kernel)
    return kern(table, idx_mat)


def _sc_scatter_kernel(src_hbm, pos_hbm, out_hbm, pos_v, rows_v, sem):
    wid = lax.axis_index("s") * SC_CORES + lax.axis_index("c")
    base = wid * ROWS_PER_WORKER
    pltpu.sync_copy(pos_hbm.at[pl.ds(wid * N_CHUNKS, N_CHUNKS)], pos_v)
    for h in range(2):
        pltpu.sync_copy(
            src_hbm.at[pl.ds(base + h * 2 * GATHER_CHUNK, 2 * GATHER_CHUNK)],
            rows_v)
        for c in range(2):
            pltpu.async_copy(
                rows_v.at[pl.ds(c * GATHER_CHUNK, GATHER_CHUNK)],
                out_hbm.at[pos_v.at[h * 2 + c]],
                sem,
            ).wait()


def _sc_scatter(src, pos_mat):
    mesh = plsc.VectorSubcoreMesh(core_axis_name="c", subcore_axis_name="s")
    kern = functools.partial(
        pl.kernel,
        mesh=mesh,
        out_type=jax.ShapeDtypeStruct((N_TOK, ST_W), jnp.float32),
        scratch_types=[
            pltpu.VMEM((N_CHUNKS, GATHER_CHUNK), jnp.int32),
            pltpu.VMEM((2 * GATHER_CHUNK, ST_W), jnp.float32),
            pltpu.SemaphoreType.DMA,
        ],
    )(_sc_scatter_kernel)
    return kern(src, pos_mat)


def _p3_body(idx_ref, g_ref, vals_ref, read_ref, cnt_ref,
             st_ref, pos_ref, jlo_ref, jhi_ref):
    i = pl.program_id(0)
    cnt = cnt_ref[...]
    iota_r = lax.broadcasted_iota(jnp.int32, (N_CMB, 1), 0)
    below = jnp.where(iota_r < i, 1.0, 0.0)
    prior = jnp.sum(cnt * below, axis=0, keepdims=True)
    colsum = jnp.sum(cnt, axis=0, keepdims=True)
    iota_b = lax.broadcasted_iota(jnp.int32, (1, N_BUCKETS), 1)
    incl = colsum
    sh = 1
    while sh < N_BUCKETS:
        rolled = pltpu.roll(incl, sh, 1)
        incl = incl + jnp.where(iota_b >= sh, rolled, 0.0)
        sh *= 2
    off = jnp.where(iota_b >= 1, pltpu.roll(incl, 1, 1), 0.0)
    base_row = off + prior
    rowstart = (512 * i).astype(jnp.float32)
    hit = jnp.logical_and(off < rowstart + 512.0,
                          off + colsum > rowstart)
    kjlo = jnp.floor(off * (1.0 / 512.0))
    kjhi = jnp.floor((off + colsum - 1.0) * (1.0 / 512.0))
    jlo = jnp.min(jnp.where(hit, kjlo, float(N_CMB)), axis=1, keepdims=True)
    jhi = jnp.max(jnp.where(hit, kjhi, -1.0), axis=1, keepdims=True)
    jlo_ref[...] = jlo.astype(jnp.int32).reshape(1, 1, 1)
    jhi_ref[...] = jhi.astype(jnp.int32).reshape(1, 1, 1)

    idx = idx_ref[...]
    d = g_ref[...] * (vals_ref[...] - read_ref[...])
    b = idx >> BUCKET_SHIFT
    iota_b = lax.broadcasted_iota(jnp.int32, (1, N_BUCKETS), 1)
    oh_f = jnp.where(b == iota_b, 1.0, 0.0)
    oh_b = oh_f.astype(jnp.bfloat16)
    r_io = lax.broadcasted_iota(jnp.int32, (CMB_BLK, 1), 0)
    c_io = lax.broadcasted_iota(jnp.int32, (1, CMB_BLK), 1)
    ls = jnp.where(r_io > c_io, 1.0, 0.0).astype(jnp.bfloat16)
    rank_mat = jnp.dot(ls, oh_b, preferred_element_type=jnp.float32)
    rank = jnp.sum(rank_mat * oh_f, axis=1, keepdims=True)
    base_sel = jnp.sum(base_row * oh_f, axis=1, keepdims=True)
    pos_ref[...] = (base_sel + rank).astype(jnp.int32)
    iota_t = lax.broadcasted_iota(jnp.int32, (1, N_BITS), 1)
    tbits = jnp.where(jnp.bitwise_and(idx >> iota_t, 1) == 1, 1.0, 0.0)
    s = jnp.sum(tbits, axis=1, keepdims=True)
    pad = jnp.zeros((CMB_BLK, ST_W - D - 1 - N_BITS - 1), jnp.float32)
    st_ref[...] = jnp.concatenate(
        [d, idx.astype(jnp.float32), tbits, s, pad], axis=1)


def _p3(idx, g, vals, read, cnt):
    return pl.pallas_call(
        _p3_body,
        grid=(N_CMB,),
        in_specs=[
            pl.BlockSpec((CMB_BLK, 1), lambda i: (i, 0)),
            pl.BlockSpec((CMB_BLK, 1), lambda i: (i, 0)),
            pl.BlockSpec((CMB_BLK, D), lambda i: (i, 0)),
            pl.BlockSpec((CMB_BLK, D), lambda i: (i, 0)),
            pl.BlockSpec((N_CMB, N_BUCKETS), lambda i: (0, 0)),
        ],
        out_specs=[
            pl.BlockSpec((CMB_BLK, ST_W), lambda i: (i, 0)),
            pl.BlockSpec((CMB_BLK, 1), lambda i: (i, 0)),
            pl.BlockSpec((1, 1, 1), lambda i: (i, 0, 0)),
            pl.BlockSpec((1, 1, 1), lambda i: (i, 0, 0)),
        ],
        out_shape=[
            jax.ShapeDtypeStruct((N_TOK, ST_W), jnp.float32),
            jax.ShapeDtypeStruct((N_TOK, 1), jnp.int32),
            jax.ShapeDtypeStruct((N_CMB, 1, 1), jnp.int32),
            jax.ShapeDtypeStruct((N_CMB, 1, 1), jnp.int32),
        ],
    )(idx, g, vals, read, cnt)


def _d2_body(st_ref, d2_ref, ua_ref, ub_ref):
    d = st_ref[:, :D]
    hi = d.astype(jnp.bfloat16)
    lo = (d - hi.astype(jnp.float32)).astype(jnp.bfloat16)
    d2_ref[...] = jnp.concatenate([hi, lo], axis=1)
    bits = st_ref[:, D + 1:D + 1 + N_BITS]
    s = st_ref[:, D + 1 + N_BITS:D + 2 + N_BITS]
    ua_ref[...] = jnp.concatenate(
        [2.0 * bits, jnp.ones((DENSE_BLK, 1), jnp.float32)],
        axis=1).astype(jnp.bfloat16)
    ub_ref[...] = jnp.concatenate([bits, -s], axis=1).astype(jnp.bfloat16)


def _d2(st):
    n_blk = N_TOK // DENSE_BLK
    return pl.pallas_call(
        _d2_body,
        grid=(n_blk,),
        in_specs=[pl.BlockSpec((DENSE_BLK, ST_W), lambda i: (i, 0))],
        out_specs=[
            pl.BlockSpec((DENSE_BLK, 2 * D), lambda i: (i, 0)),
            pl.BlockSpec((DENSE_BLK, N_BITS + 1), lambda i: (i, 0)),
            pl.BlockSpec((DENSE_BLK, N_BITS + 1), lambda i: (i, 0)),
        ],
        out_shape=[
            jax.ShapeDtypeStruct((N_TOK, 2 * D), jnp.bfloat16),
            jax.ShapeDtypeStruct((N_TOK, N_BITS + 1), jnp.bfloat16),
            jax.ShapeDtypeStruct((N_TOK, N_BITS + 1), jnp.bfloat16),
        ],
    )(st)


def _combine_body(jlo_ref, jhi_ref, st_ref, d2_ref, ua_ref, ub_ref,
                  corr_ref, acc_ref):
    i = pl.program_id(0)
    sti = st_ref[pl.ds(i * CMB_BLK, CMB_BLK), :]
    s_i = sti[:, D + 1 + N_BITS:D + 2 + N_BITS]
    ua_i = ua_ref[pl.ds(i * CMB_BLK, CMB_BLK), :]
    acc_ref[...] = jnp.zeros((CMB_BLK, 2 * D), jnp.float32)

    def jbody(j, _):
        ub_j = ub_ref[pl.ds(j * CMB_BLK, CMB_BLK), :]
        m = lax.dot_general(ua_i, ub_j, (((1,), (1,)), ((), ())),
                            preferred_element_type=jnp.float32)
        mask = jnp.maximum(m - s_i + 1.0, 0.0).astype(jnp.bfloat16)
        d2j = d2_ref[pl.ds(j * CMB_BLK, CMB_BLK), :]
        acc_ref[...] += jnp.dot(mask, d2j, preferred_element_type=jnp.float32)
        return 0

    lax.fori_loop(jlo_ref[i], jhi_ref[i] + 1, jbody, 0)
    acc = acc_ref[...]
    corr_ref[...] = acc[:, :D] + acc[:, D:]


def _combine(jlo, jhi, st, d2, ua, ub):
    grid_spec = pltpu.PrefetchScalarGridSpec(
        num_scalar_prefetch=2,
        grid=(N_CMB,),
        in_specs=[
            pl.BlockSpec((N_TOK, ST_W), lambda i, jlo_r, jhi_r: (0, 0)),
            pl.BlockSpec((N_TOK, 2 * D), lambda i, jlo_r, jhi_r: (0, 0)),
            pl.BlockSpec((N_TOK, N_BITS + 1), lambda i, jlo_r, jhi_r: (0, 0)),
            pl.BlockSpec((N_TOK, N_BITS + 1), lambda i, jlo_r, jhi_r: (0, 0)),
        ],
        out_specs=pl.BlockSpec((CMB_BLK, D), lambda i, jlo_r, jhi_r: (i, 0)),
        scratch_shapes=[pltpu.VMEM((CMB_BLK, 2 * D), jnp.float32)],
    )
    return pl.pallas_call(
        _combine_body,
        grid_spec=grid_spec,
        out_shape=jax.ShapeDtypeStruct((N_TOK, D), jnp.float32),
    )(jlo, jhi, st, d2, ua, ub)


def _final_body(read_ref, corr_ref, out_ref):
    out_ref[...] = read_ref[...] + corr_ref[...]


def _final(read, corr):
    n_blk = N_TOK // DENSE_BLK
    return pl.pallas_call(
        _final_body,
        grid=(n_blk,),
        in_specs=[
            pl.BlockSpec((DENSE_BLK, D), lambda i: (i, 0)),
            pl.BlockSpec((DENSE_BLK, D), lambda i: (i, 0)),
        ],
        out_specs=pl.BlockSpec((DENSE_BLK, D), lambda i: (i, 0)),
        out_shape=jax.ShapeDtypeStruct((N_TOK, D), jnp.float32),
    )(read, corr)


def kernel(x, Wk, bk, Wv, bv, Wc, bc, W1, b1, W2, b2, P, V_mem):
    WkT = Wk.T
    WvT = Wv.T
    WcT = Wc.T
    W1T = W1.T
    W2r = W2.reshape(1, -1)
    bk2 = bk.reshape(1, -1)
    bv2 = bv.reshape(1, -1)
    bc2 = bc.reshape(1, -1)
    b12 = b1.reshape(1, -1)
    b22 = b2.reshape(1, 1)

    vals, g, idx, cnt3 = _dense_proj(x, WkT, bk2, WvT, bv2, WcT, bc2,
                                     W1T, b12, W2r, b22, P)
    idx_mat = idx.reshape(SC_WORKERS * N_CHUNKS, GATHER_CHUNK)
    read = _sc_gather(V_mem, idx_mat)

    cnt = cnt3.reshape(N_CMB, N_BUCKETS)
    st_src, pos, jlo, jhi = _p3(idx, g, vals, read, cnt)
    pos_mat = pos.reshape(SC_WORKERS * N_CHUNKS, GATHER_CHUNK)
    st = _sc_scatter(st_src, pos_mat)

    d2, ua, ub = _d2(st)
    corr_perm = _combine(jlo.reshape(N_CMB), jhi.reshape(N_CMB),
                         st, d2, ua, ub)
    corr = _sc_gather(corr_perm, pos_mat)
    return _final(read, corr)


# speedup vs baseline: 1.0702x; 1.0702x over previous
import functools

import jax
import jax.numpy as jnp
from jax import lax
from jax.experimental import pallas as pl
from jax.experimental.pallas import tpu as pltpu
from jax.experimental.pallas import tpu_sc as plsc

N_TOK = 16384
D = 128
N_BITS = 17

BUCKET_SHIFT = 10
N_BUCKETS = 1 << (N_BITS - BUCKET_SHIFT)

SC_CORES = 2
SC_SUBCORES = 16
SC_WORKERS = SC_CORES * SC_SUBCORES
ROWS_PER_WORKER = N_TOK // SC_WORKERS
GATHER_CHUNK = 128
N_CHUNKS = ROWS_PER_WORKER // GATHER_CHUNK

DENSE_BLK = 1024
CMB_BLK = 512
N_CMB = N_TOK // CMB_BLK
ST_W = 256


def _dense_body(x_ref, wkT_ref, bk_ref, wvT_ref, bv_ref, wcT_ref, bc_ref,
                w1T_ref, b1_ref, w2_ref, b2_ref, p_ref,
                vals_ref, g_ref, idx_ref, cnt_ref):
    xb = x_ref[...]
    keys = jnp.dot(xb, wkT_ref[...],
                   preferred_element_type=jnp.float32) + bk_ref[...]
    vals = jnp.dot(xb, wvT_ref[...],
                   preferred_element_type=jnp.float32) + bv_ref[...]
    ctx = jnp.dot(xb, wcT_ref[...],
                  preferred_element_type=jnp.float32) + bc_ref[...]
    h = jax.nn.relu(jnp.dot(ctx, w1T_ref[...],
                            preferred_element_type=jnp.float32) + b1_ref[...])
    glogit = jnp.sum(h * w2_ref[...], axis=1, keepdims=True) + b2_ref[...]
    g = jax.nn.sigmoid(glogit)
    s = jnp.dot(keys, p_ref[...],
                preferred_element_type=jnp.float32)
    bits = (s > 0.0).astype(jnp.int32)
    powers = jnp.left_shift(
        1, lax.broadcasted_iota(jnp.int32, (1, N_BITS), 1))
    idx = jnp.sum(bits * powers, axis=1, keepdims=True)
    vals_ref[...] = vals
    g_ref[...] = g
    idx_ref[...] = idx
    b = idx >> BUCKET_SHIFT
    iota_b = lax.broadcasted_iota(jnp.int32, (1, N_BUCKETS), 1)
    oh = jnp.where(b == iota_b, 1.0, 0.0)
    c0 = jnp.sum(oh[:CMB_BLK], axis=0, keepdims=True)
    c1 = jnp.sum(oh[CMB_BLK:], axis=0, keepdims=True)
    cnt_ref[...] = jnp.concatenate([c0, c1], axis=0).reshape(1, 2, N_BUCKETS)


def _dense_proj(x, WkT, bk, WvT, bv, WcT, bc, W1T, b1, W2r, b2, P):
    n_blk = N_TOK // DENSE_BLK
    full = lambda shape: pl.BlockSpec(shape, lambda i: (0, 0))
    return pl.pallas_call(
        _dense_body,
        grid=(n_blk,),
        in_specs=[
            pl.BlockSpec((DENSE_BLK, D), lambda i: (i, 0)),
            full(WkT.shape), full(bk.shape),
            full(WvT.shape), full(bv.shape),
            full(WcT.shape), full(bc.shape),
            full(W1T.shape), full(b1.shape),
            full(W2r.shape), full(b2.shape),
            full(P.shape),
        ],
        out_specs=[
            pl.BlockSpec((DENSE_BLK, D), lambda i: (i, 0)),
            pl.BlockSpec((DENSE_BLK, 1), lambda i: (i, 0)),
            pl.BlockSpec((DENSE_BLK, 1), lambda i: (i, 0)),
            pl.BlockSpec((1, 2, N_BUCKETS), lambda i: (i, 0, 0)),
        ],
        out_shape=[
            jax.ShapeDtypeStruct((N_TOK, D), jnp.float32),
            jax.ShapeDtypeStruct((N_TOK, 1), jnp.float32),
            jax.ShapeDtypeStruct((N_TOK, 1), jnp.int32),
            jax.ShapeDtypeStruct((n_blk, 2, N_BUCKETS), jnp.float32),
        ],
    )(x, WkT, bk, WvT, bv, WcT, bc, W1T, b1, W2r, b2, P)


def _sc_gather_kernel(table_hbm, idx_hbm, out_hbm, idx_v, rows_v, sem):
    wid = lax.axis_index("s") * SC_CORES + lax.axis_index("c")
    base = wid * ROWS_PER_WORKER
    pltpu.sync_copy(idx_hbm.at[pl.ds(wid * N_CHUNKS, N_CHUNKS)], idx_v)
    for c in range(N_CHUNKS):
        pltpu.async_copy(
            table_hbm.at[idx_v.at[c]],
            rows_v.at[pl.ds(c * GATHER_CHUNK, GATHER_CHUNK)],
            sem,
        ).wait()
    pltpu.sync_copy(rows_v, out_hbm.at[pl.ds(base, ROWS_PER_WORKER)])


def _sc_gather(table, idx_mat):
    mesh = plsc.VectorSubcoreMesh(core_axis_name="c", subcore_axis_name="s")
    kern = functools.partial(
        pl.kernel,
        mesh=mesh,
        out_type=jax.ShapeDtypeStruct((N_TOK, D), jnp.float32),
        scratch_types=[
            pltpu.VMEM((N_CHUNKS, GATHER_CHUNK), jnp.int32),
            pltpu.VMEM((ROWS_PER_WORKER, D), jnp.float32),
            pltpu.SemaphoreType.DMA,
        ],
    )(_sc_gather_kernel)
    return kern(table, idx_mat)


def _sc_scatter_kernel(src_hbm, pos_hbm, out_hbm, pos_v, rows_v, sem):
    wid = lax.axis_index("s") * SC_CORES + lax.axis_index("c")
    base = wid * ROWS_PER_WORKER
    pltpu.sync_copy(pos_hbm.at[pl.ds(wid * N_CHUNKS, N_CHUNKS)], pos_v)
    for h in range(2):
        pltpu.sync_copy(
            src_hbm.at[pl.ds(base + h * 2 * GATHER_CHUNK, 2 * GATHER_CHUNK)],
            rows_v)
        for c in range(2):
            pltpu.async_copy(
                rows_v.at[pl.ds(c * GATHER_CHUNK, GATHER_CHUNK)],
                out_hbm.at[pos_v.at[h * 2 + c]],
                sem,
            ).wait()


def _sc_scatter(src, pos_mat):
    mesh = plsc.VectorSubcoreMesh(core_axis_name="c", subcore_axis_name="s")
    kern = functools.partial(
        pl.kernel,
        mesh=mesh,
        out_type=jax.ShapeDtypeStruct((N_TOK, ST_W), jnp.float32),
        scratch_types=[
            pltpu.VMEM((N_CHUNKS, GATHER_CHUNK), jnp.int32),
            pltpu.VMEM((2 * GATHER_CHUNK, ST_W), jnp.float32),
            pltpu.SemaphoreType.DMA,
        ],
    )(_sc_scatter_kernel)
    return kern(src, pos_mat)


def _p2_body(cnt_ref, base_ref, jlo_ref, jhi_ref):
    def row(b, acc):
        base_ref[pl.ds(b, 1), :, :] = acc.reshape(1, 1, N_BUCKETS)
        return acc + cnt_ref[pl.ds(b, 1), :]

    colsum = lax.fori_loop(0, N_CMB, row,
                           jnp.zeros((1, N_BUCKETS), jnp.float32))
    iota_b = lax.broadcasted_iota(jnp.int32, (1, N_BUCKETS), 1)
    incl = colsum
    sh = 1
    while sh < N_BUCKETS:
        rolled = pltpu.roll(incl, sh, 1)
        incl = incl + jnp.where(iota_b >= sh, rolled, 0.0)
        sh *= 2
    off = jnp.where(iota_b >= 1, pltpu.roll(incl, 1, 1), 0.0)
    base_ref[...] += off.reshape(1, 1, N_BUCKETS)
    rowstart = 512.0 * lax.broadcasted_iota(
        jnp.int32, (N_CMB, 1), 0).astype(jnp.float32)
    hit = jnp.logical_and(off < rowstart + 512.0,
                          off + colsum > rowstart)
    kjlo = jnp.floor(off * (1.0 / 512.0))
    kjhi = jnp.floor((off + colsum - 1.0) * (1.0 / 512.0))
    jlo = jnp.min(jnp.where(hit, kjlo, float(N_CMB)), axis=1, keepdims=True)
    jhi = jnp.max(jnp.where(hit, kjhi, -1.0), axis=1, keepdims=True)
    jlo_ref[...] = jlo.astype(jnp.int32)
    jhi_ref[...] = jhi.astype(jnp.int32)


def _p2(cnt):
    return pl.pallas_call(
        _p2_body,
        grid=(1,),
        in_specs=[pl.BlockSpec((N_CMB, N_BUCKETS), lambda i: (0, 0))],
        out_specs=[
            pl.BlockSpec((N_CMB, 1, N_BUCKETS), lambda i: (0, 0, 0)),
            pl.BlockSpec((N_CMB, 1), lambda i: (0, 0)),
            pl.BlockSpec((N_CMB, 1), lambda i: (0, 0)),
        ],
        out_shape=[
            jax.ShapeDtypeStruct((N_CMB, 1, N_BUCKETS), jnp.float32),
            jax.ShapeDtypeStruct((N_CMB, 1), jnp.int32),
            jax.ShapeDtypeStruct((N_CMB, 1), jnp.int32),
        ],
    )(cnt)


def _p3_body(idx_ref, g_ref, vals_ref, read_ref, base_ref, st_ref, pos_ref):
    idx = idx_ref[...]
    d = g_ref[...] * (vals_ref[...] - read_ref[...])
    b = idx >> BUCKET_SHIFT
    iota_b = lax.broadcasted_iota(jnp.int32, (1, N_BUCKETS), 1)
    oh_f = jnp.where(b == iota_b, 1.0, 0.0)
    oh_b = oh_f.astype(jnp.bfloat16)
    r_io = lax.broadcasted_iota(jnp.int32, (CMB_BLK, 1), 0)
    c_io = lax.broadcasted_iota(jnp.int32, (1, CMB_BLK), 1)
    ls = jnp.where(r_io > c_io, 1.0, 0.0).astype(jnp.bfloat16)
    rank_mat = jnp.dot(ls, oh_b, preferred_element_type=jnp.float32)
    rank = jnp.sum(rank_mat * oh_f, axis=1, keepdims=True)
    base_row = base_ref[...].reshape(1, N_BUCKETS)
    base_sel = jnp.sum(base_row * oh_f, axis=1, keepdims=True)
    pos_ref[...] = (base_sel + rank).astype(jnp.int32)
    pad = jnp.zeros((CMB_BLK, ST_W - D - 1), jnp.float32)
    st_ref[...] = jnp.concatenate([d, idx.astype(jnp.float32), pad], axis=1)


def _p3(idx, g, vals, read, base):
    return pl.pallas_call(
        _p3_body,
        grid=(N_CMB,),
        in_specs=[
            pl.BlockSpec((CMB_BLK, 1), lambda i: (i, 0)),
            pl.BlockSpec((CMB_BLK, 1), lambda i: (i, 0)),
            pl.BlockSpec((CMB_BLK, D), lambda i: (i, 0)),
            pl.BlockSpec((CMB_BLK, D), lambda i: (i, 0)),
            pl.BlockSpec((1, 1, N_BUCKETS), lambda i: (i, 0, 0)),
        ],
        out_specs=[
            pl.BlockSpec((CMB_BLK, ST_W), lambda i: (i, 0)),
            pl.BlockSpec((CMB_BLK, 1), lambda i: (i, 0)),
        ],
        out_shape=[
            jax.ShapeDtypeStruct((N_TOK, ST_W), jnp.float32),
            jax.ShapeDtypeStruct((N_TOK, 1), jnp.int32),
        ],
    )(idx, g, vals, read, base)


def _bits_of(idx_f):
    ii = idx_f.astype(jnp.int32)
    iota = lax.broadcasted_iota(jnp.int32, (1, N_BITS), 1)
    bits = jnp.where(jnp.bitwise_and(ii >> iota, 1) == 1, 1.0, 0.0)
    return bits, jnp.sum(bits, axis=1, keepdims=True)


def _combine_body(jlo_ref, jhi_ref, st_ref, corr_ref, acc_ref):
    i = pl.program_id(0)
    sti = st_ref[pl.ds(i * CMB_BLK, CMB_BLK), :]
    bits_i, s_i = _bits_of(sti[:, D:D + 1])
    ua_i = jnp.concatenate(
        [2.0 * bits_i, jnp.ones((CMB_BLK, 1), jnp.float32)],
        axis=1).astype(jnp.bfloat16)
    acc_ref[...] = jnp.zeros((CMB_BLK, 2 * D), jnp.float32)

    def jbody(j, _):
        stj = st_ref[pl.ds(j * CMB_BLK, CMB_BLK), :]
        bits_j, s_j = _bits_of(stj[:, D:D + 1])
        ub_j = jnp.concatenate([bits_j, -s_j], axis=1).astype(jnp.bfloat16)
        m = lax.dot_general(ua_i, ub_j, (((1,), (1,)), ((), ())),
                            preferred_element_type=jnp.float32)
        mask = jnp.maximum(m - s_i + 1.0, 0.0).astype(jnp.bfloat16)
        dj = stj[:, :D]
        hi = dj.astype(jnp.bfloat16)
        lo = (dj - hi.astype(jnp.float32)).astype(jnp.bfloat16)
        d2 = jnp.concatenate([hi, lo], axis=1)
        acc_ref[...] += jnp.dot(mask, d2, preferred_element_type=jnp.float32)
        return 0

    lax.fori_loop(jlo_ref[i], jhi_ref[i] + 1, jbody, 0)
    acc = acc_ref[...]
    corr_ref[...] = acc[:, :D] + acc[:, D:]


def _combine(jlo, jhi, st):
    grid_spec = pltpu.PrefetchScalarGridSpec(
        num_scalar_prefetch=2,
        grid=(N_CMB,),
        in_specs=[pl.BlockSpec((N_TOK, ST_W), lambda i, jlo_r, jhi_r: (0, 0))],
        out_specs=pl.BlockSpec((CMB_BLK, D), lambda i, jlo_r, jhi_r: (i, 0)),
        scratch_shapes=[pltpu.VMEM((CMB_BLK, 2 * D), jnp.float32)],
    )
    return pl.pallas_call(
        _combine_body,
        grid_spec=grid_spec,
        out_shape=jax.ShapeDtypeStruct((N_TOK, D), jnp.float32),
    )(jlo, jhi, st)


def _final_body(read_ref, corr_ref, out_ref):
    out_ref[...] = read_ref[...] + corr_ref[...]


def _final(read, corr):
    n_blk = N_TOK // DENSE_BLK
    return pl.pallas_call(
        _final_body,
        grid=(n_blk,),
        in_specs=[
            pl.BlockSpec((DENSE_BLK, D), lambda i: (i, 0)),
            pl.BlockSpec((DENSE_BLK, D), lambda i: (i, 0)),
        ],
        out_specs=pl.BlockSpec((DENSE_BLK, D), lambda i: (i, 0)),
        out_shape=jax.ShapeDtypeStruct((N_TOK, D), jnp.float32),
    )(read, corr)


def kernel(x, Wk, bk, Wv, bv, Wc, bc, W1, b1, W2, b2, P, V_mem):
    WkT = Wk.T
    WvT = Wv.T
    WcT = Wc.T
    W1T = W1.T
    W2r = W2.reshape(1, -1)
    bk2 = bk.reshape(1, -1)
    bv2 = bv.reshape(1, -1)
    bc2 = bc.reshape(1, -1)
    b12 = b1.reshape(1, -1)
    b22 = b2.reshape(1, 1)

    vals, g, idx, cnt3 = _dense_proj(x, WkT, bk2, WvT, bv2, WcT, bc2,
                                     W1T, b12, W2r, b22, P)
    idx_mat = idx.reshape(SC_WORKERS * N_CHUNKS, GATHER_CHUNK)
    read = _sc_gather(V_mem, idx_mat)

    cnt = cnt3.reshape(N_CMB, N_BUCKETS)
    base, jlo, jhi = _p2(cnt)
    st_src, pos = _p3(idx, g, vals, read, base)
    pos_mat = pos.reshape(SC_WORKERS * N_CHUNKS, GATHER_CHUNK)
    st = _sc_scatter(st_src, pos_mat)

    corr_perm = _combine(jlo.reshape(N_CMB), jhi.reshape(N_CMB), st)
    corr = _sc_gather(corr_perm, pos_mat)
    return _final(read, corr)


# speedup vs baseline: 1.1133x; 1.0402x over previous
import functools

import jax
import jax.numpy as jnp
from jax import lax
from jax.experimental import pallas as pl
from jax.experimental.pallas import tpu as pltpu
from jax.experimental.pallas import tpu_sc as plsc

N_TOK = 16384
D = 128
N_BITS = 17

BUCKET_SHIFT = 10
N_BUCKETS = 1 << (N_BITS - BUCKET_SHIFT)

SC_CORES = 2
SC_SUBCORES = 16
SC_WORKERS = SC_CORES * SC_SUBCORES
ROWS_PER_WORKER = N_TOK // SC_WORKERS
GATHER_CHUNK = 128
N_CHUNKS = ROWS_PER_WORKER // GATHER_CHUNK

DENSE_BLK = 1024
CMB_BLK = 512
N_CMB = N_TOK // CMB_BLK
ST_W = 256


def _dense_body(x_ref, wkT_ref, bk_ref, wvT_ref, bv_ref, wcT_ref, bc_ref,
                w1T_ref, b1_ref, w2_ref, b2_ref, p_ref,
                vals_ref, g_ref, idx_ref, cnt_ref):
    xb = x_ref[...]
    keys = jnp.dot(xb, wkT_ref[...],
                   preferred_element_type=jnp.float32) + bk_ref[...]
    vals = jnp.dot(xb, wvT_ref[...],
                   preferred_element_type=jnp.float32) + bv_ref[...]
    ctx = jnp.dot(xb, wcT_ref[...],
                  preferred_element_type=jnp.float32) + bc_ref[...]
    h = jax.nn.relu(jnp.dot(ctx, w1T_ref[...],
                            preferred_element_type=jnp.float32) + b1_ref[...])
    glogit = jnp.sum(h * w2_ref[...], axis=1, keepdims=True) + b2_ref[...]
    g = jax.nn.sigmoid(glogit)
    s = jnp.dot(keys, p_ref[...],
                preferred_element_type=jnp.float32)
    bits = (s > 0.0).astype(jnp.int32)
    powers = jnp.left_shift(
        1, lax.broadcasted_iota(jnp.int32, (1, N_BITS), 1))
    idx = jnp.sum(bits * powers, axis=1, keepdims=True)
    vals_ref[...] = vals
    g_ref[...] = g
    idx_ref[...] = idx
    b = idx >> BUCKET_SHIFT
    iota_b = lax.broadcasted_iota(jnp.int32, (1, N_BUCKETS), 1)
    oh = jnp.where(b == iota_b, 1.0, 0.0)
    c0 = jnp.sum(oh[:CMB_BLK], axis=0, keepdims=True)
    c1 = jnp.sum(oh[CMB_BLK:], axis=0, keepdims=True)
    cnt_ref[...] = jnp.concatenate([c0, c1], axis=0).reshape(1, 2, N_BUCKETS)


def _dense_proj(x, WkT, bk, WvT, bv, WcT, bc, W1T, b1, W2r, b2, P):
    n_blk = N_TOK // DENSE_BLK
    full = lambda shape: pl.BlockSpec(shape, lambda i: (0, 0))
    return pl.pallas_call(
        _dense_body,
        grid=(n_blk,),
        in_specs=[
            pl.BlockSpec((DENSE_BLK, D), lambda i: (i, 0)),
            full(WkT.shape), full(bk.shape),
            full(WvT.shape), full(bv.shape),
            full(WcT.shape), full(bc.shape),
            full(W1T.shape), full(b1.shape),
            full(W2r.shape), full(b2.shape),
            full(P.shape),
        ],
        out_specs=[
            pl.BlockSpec((DENSE_BLK, D), lambda i: (i, 0)),
            pl.BlockSpec((DENSE_BLK, 1), lambda i: (i, 0)),
            pl.BlockSpec((DENSE_BLK, 1), lambda i: (i, 0)),
            pl.BlockSpec((1, 2, N_BUCKETS), lambda i: (i, 0, 0)),
        ],
        out_shape=[
            jax.ShapeDtypeStruct((N_TOK, D), jnp.float32),
            jax.ShapeDtypeStruct((N_TOK, 1), jnp.float32),
            jax.ShapeDtypeStruct((N_TOK, 1), jnp.int32),
            jax.ShapeDtypeStruct((n_blk, 2, N_BUCKETS), jnp.float32),
        ],
    )(x, WkT, bk, WvT, bv, WcT, bc, W1T, b1, W2r, b2, P)


def _sc_gather_kernel(table_hbm, idx_hbm, out_hbm, idx_v, rows_v, sem):
    wid = lax.axis_index("s") * SC_CORES + lax.axis_index("c")
    base = wid * ROWS_PER_WORKER
    pltpu.sync_copy(idx_hbm.at[pl.ds(wid * N_CHUNKS, N_CHUNKS)], idx_v)
    for c in range(N_CHUNKS):
        pltpu.async_copy(
            table_hbm.at[idx_v.at[c]],
            rows_v.at[pl.ds(c * GATHER_CHUNK, GATHER_CHUNK)],
            sem,
        ).wait()
    pltpu.sync_copy(rows_v, out_hbm.at[pl.ds(base, ROWS_PER_WORKER)])


def _sc_gather(table, idx_mat):
    mesh = plsc.VectorSubcoreMesh(core_axis_name="c", subcore_axis_name="s")
    kern = functools.partial(
        pl.kernel,
        mesh=mesh,
        out_type=jax.ShapeDtypeStruct((N_TOK, D), jnp.float32),
        scratch_types=[
            pltpu.VMEM((N_CHUNKS, GATHER_CHUNK), jnp.int32),
            pltpu.VMEM((ROWS_PER_WORKER, D), jnp.float32),
            pltpu.SemaphoreType.DMA,
        ],
    )(_sc_gather_kernel)
    return kern(table, idx_mat)


def _sc_stage_kernel(table_hbm, idx_hbm, src_hbm, pos_hbm,
                     st_hbm, rdp_hbm, idx_v, pos_v, rows_v, st_v, sem):
    wid = lax.axis_index("s") * SC_CORES + lax.axis_index("c")
    base = wid * ROWS_PER_WORKER
    pltpu.sync_copy(idx_hbm.at[pl.ds(wid * N_CHUNKS, N_CHUNKS)], idx_v)
    pltpu.sync_copy(pos_hbm.at[pl.ds(wid * N_CHUNKS, N_CHUNKS)], pos_v)
    for h in range(2):
        for c in range(2):
            pltpu.async_copy(
                table_hbm.at[idx_v.at[h * 2 + c]],
                rows_v.at[pl.ds(c * GATHER_CHUNK, GATHER_CHUNK)],
                sem,
            ).wait()
        for c in range(2):
            pltpu.async_copy(
                rows_v.at[pl.ds(c * GATHER_CHUNK, GATHER_CHUNK)],
                rdp_hbm.at[pos_v.at[h * 2 + c]],
                sem,
            ).wait()
        pltpu.sync_copy(
            src_hbm.at[pl.ds(base + h * 2 * GATHER_CHUNK, 2 * GATHER_CHUNK)],
            st_v)
        for c in range(2):
            pltpu.async_copy(
                st_v.at[pl.ds(c * GATHER_CHUNK, GATHER_CHUNK)],
                st_hbm.at[pos_v.at[h * 2 + c]],
                sem,
            ).wait()


def _sc_stage(V_mem, idx_mat, src, pos_mat):
    mesh = plsc.VectorSubcoreMesh(core_axis_name="c", subcore_axis_name="s")
    kern = functools.partial(
        pl.kernel,
        mesh=mesh,
        out_type=[
            jax.ShapeDtypeStruct((N_TOK, ST_W), jnp.float32),
            jax.ShapeDtypeStruct((N_TOK, D), jnp.float32),
        ],
        scratch_types=[
            pltpu.VMEM((N_CHUNKS, GATHER_CHUNK), jnp.int32),
            pltpu.VMEM((N_CHUNKS, GATHER_CHUNK), jnp.int32),
            pltpu.VMEM((2 * GATHER_CHUNK, D), jnp.float32),
            pltpu.VMEM((2 * GATHER_CHUNK, ST_W), jnp.float32),
            pltpu.SemaphoreType.DMA,
        ],
    )(_sc_stage_kernel)
    return kern(V_mem, idx_mat, src, pos_mat)


def _p2_body(cnt_ref, base_ref, jlo_ref, jhi_ref):
    def row(b, acc):
        base_ref[pl.ds(b, 1), :, :] = acc.reshape(1, 1, N_BUCKETS)
        return acc + cnt_ref[pl.ds(b, 1), :]

    colsum = lax.fori_loop(0, N_CMB, row,
                           jnp.zeros((1, N_BUCKETS), jnp.float32))
    iota_b = lax.broadcasted_iota(jnp.int32, (1, N_BUCKETS), 1)
    incl = colsum
    sh = 1
    while sh < N_BUCKETS:
        rolled = pltpu.roll(incl, sh, 1)
        incl = incl + jnp.where(iota_b >= sh, rolled, 0.0)
        sh *= 2
    off = jnp.where(iota_b >= 1, pltpu.roll(incl, 1, 1), 0.0)
    base_ref[...] += off.reshape(1, 1, N_BUCKETS)
    rowstart = 512.0 * lax.broadcasted_iota(
        jnp.int32, (N_CMB, 1), 0).astype(jnp.float32)
    hit = jnp.logical_and(off < rowstart + 512.0,
                          off + colsum > rowstart)
    kjlo = jnp.floor(off * (1.0 / 512.0))
    kjhi = jnp.floor((off + colsum - 1.0) * (1.0 / 512.0))
    jlo = jnp.min(jnp.where(hit, kjlo, float(N_CMB)), axis=1, keepdims=True)
    jhi = jnp.max(jnp.where(hit, kjhi, -1.0), axis=1, keepdims=True)
    jlo_ref[...] = jlo.astype(jnp.int32)
    jhi_ref[...] = jhi.astype(jnp.int32)


def _p2(cnt):
    return pl.pallas_call(
        _p2_body,
        grid=(1,),
        in_specs=[pl.BlockSpec((N_CMB, N_BUCKETS), lambda i: (0, 0))],
        out_specs=[
            pl.BlockSpec((N_CMB, 1, N_BUCKETS), lambda i: (0, 0, 0)),
            pl.BlockSpec((N_CMB, 1), lambda i: (0, 0)),
            pl.BlockSpec((N_CMB, 1), lambda i: (0, 0)),
        ],
        out_shape=[
            jax.ShapeDtypeStruct((N_CMB, 1, N_BUCKETS), jnp.float32),
            jax.ShapeDtypeStruct((N_CMB, 1), jnp.int32),
            jax.ShapeDtypeStruct((N_CMB, 1), jnp.int32),
        ],
    )(cnt)


def _p3_body(idx_ref, g_ref, vals_ref, base_ref, st_ref, pos_ref):
    idx = idx_ref[...]
    b = idx >> BUCKET_SHIFT
    iota_b = lax.broadcasted_iota(jnp.int32, (1, N_BUCKETS), 1)
    oh_f = jnp.where(b == iota_b, 1.0, 0.0)
    oh_b = oh_f.astype(jnp.bfloat16)
    r_io = lax.broadcasted_iota(jnp.int32, (CMB_BLK, 1), 0)
    c_io = lax.broadcasted_iota(jnp.int32, (1, CMB_BLK), 1)
    ls = jnp.where(r_io > c_io, 1.0, 0.0).astype(jnp.bfloat16)
    rank_mat = jnp.dot(ls, oh_b, preferred_element_type=jnp.float32)
    rank = jnp.sum(rank_mat * oh_f, axis=1, keepdims=True)
    base_row = base_ref[...].reshape(1, N_BUCKETS)
    base_sel = jnp.sum(base_row * oh_f, axis=1, keepdims=True)
    pos_ref[...] = (base_sel + rank).astype(jnp.int32)
    pad = jnp.zeros((CMB_BLK, ST_W - D - 2), jnp.float32)
    st_ref[...] = jnp.concatenate(
        [vals_ref[...], idx.astype(jnp.float32), g_ref[...], pad], axis=1)


def _p3(idx, g, vals, base):
    return pl.pallas_call(
        _p3_body,
        grid=(N_CMB,),
        in_specs=[
            pl.BlockSpec((CMB_BLK, 1), lambda i: (i, 0)),
            pl.BlockSpec((CMB_BLK, 1), lambda i: (i, 0)),
            pl.BlockSpec((CMB_BLK, D), lambda i: (i, 0)),
            pl.BlockSpec((1, 1, N_BUCKETS), lambda i: (i, 0, 0)),
        ],
        out_specs=[
            pl.BlockSpec((CMB_BLK, ST_W), lambda i: (i, 0)),
            pl.BlockSpec((CMB_BLK, 1), lambda i: (i, 0)),
        ],
        out_shape=[
            jax.ShapeDtypeStruct((N_TOK, ST_W), jnp.float32),
            jax.ShapeDtypeStruct((N_TOK, 1), jnp.int32),
        ],
    )(idx, g, vals, base)


def _bits_of(idx_f):
    ii = idx_f.astype(jnp.int32)
    iota = lax.broadcasted_iota(jnp.int32, (1, N_BITS), 1)
    bits = jnp.where(jnp.bitwise_and(ii >> iota, 1) == 1, 1.0, 0.0)
    return bits, jnp.sum(bits, axis=1, keepdims=True)


def _combine_body(jlo_ref, jhi_ref, st_ref, rdp_ref, out_ref, acc_ref):
    i = pl.program_id(0)
    sti = st_ref[pl.ds(i * CMB_BLK, CMB_BLK), :]
    bits_i, s_i = _bits_of(sti[:, D:D + 1])
    ua_i = jnp.concatenate(
        [2.0 * bits_i, jnp.ones((CMB_BLK, 1), jnp.float32)],
        axis=1).astype(jnp.bfloat16)
    acc_ref[...] = jnp.zeros((CMB_BLK, 2 * D), jnp.float32)

    def jbody(j, _):
        stj = st_ref[pl.ds(j * CMB_BLK, CMB_BLK), :]
        bits_j, s_j = _bits_of(stj[:, D:D + 1])
        ub_j = jnp.concatenate([bits_j, -s_j], axis=1).astype(jnp.bfloat16)
        m = lax.dot_general(ua_i, ub_j, (((1,), (1,)), ((), ())),
                            preferred_element_type=jnp.float32)
        mask = jnp.maximum(m - s_i + 1.0, 0.0).astype(jnp.bfloat16)
        dj = stj[:, D + 1:D + 2] * (
            stj[:, :D] - rdp_ref[pl.ds(j * CMB_BLK, CMB_BLK), :])
        hi = dj.astype(jnp.bfloat16)
        lo = (dj - hi.astype(jnp.float32)).astype(jnp.bfloat16)
        d2 = jnp.concatenate([hi, lo], axis=1)
        acc_ref[...] += jnp.dot(mask, d2, preferred_element_type=jnp.float32)
        return 0

    lax.fori_loop(jlo_ref[i], jhi_ref[i] + 1, jbody, 0)
    acc = acc_ref[...]
    out_ref[...] = (rdp_ref[pl.ds(i * CMB_BLK, CMB_BLK), :]
                    + acc[:, :D] + acc[:, D:])


def _combine(jlo, jhi, st, rdp):
    grid_spec = pltpu.PrefetchScalarGridSpec(
        num_scalar_prefetch=2,
        grid=(N_CMB,),
        in_specs=[
            pl.BlockSpec((N_TOK, ST_W), lambda i, jlo_r, jhi_r: (0, 0)),
            pl.BlockSpec((N_TOK, D), lambda i, jlo_r, jhi_r: (0, 0)),
        ],
        out_specs=pl.BlockSpec((CMB_BLK, D), lambda i, jlo_r, jhi_r: (i, 0)),
        scratch_shapes=[pltpu.VMEM((CMB_BLK, 2 * D), jnp.float32)],
    )
    return pl.pallas_call(
        _combine_body,
        grid_spec=grid_spec,
        out_shape=jax.ShapeDtypeStruct((N_TOK, D), jnp.float32),
    )(jlo, jhi, st, rdp)


def kernel(x, Wk, bk, Wv, bv, Wc, bc, W1, b1, W2, b2, P, V_mem):
    WkT = Wk.T
    WvT = Wv.T
    WcT = Wc.T
    W1T = W1.T
    W2r = W2.reshape(1, -1)
    bk2 = bk.reshape(1, -1)
    bv2 = bv.reshape(1, -1)
    bc2 = bc.reshape(1, -1)
    b12 = b1.reshape(1, -1)
    b22 = b2.reshape(1, 1)

    vals, g, idx, cnt3 = _dense_proj(x, WkT, bk2, WvT, bv2, WcT, bc2,
                                     W1T, b12, W2r, b22, P)
    idx_mat = idx.reshape(SC_WORKERS * N_CHUNKS, GATHER_CHUNK)

    cnt = cnt3.reshape(N_CMB, N_BUCKETS)
    base, jlo, jhi = _p2(cnt)
    st_src, pos = _p3(idx, g, vals, base)
    pos_mat = pos.reshape(SC_WORKERS * N_CHUNKS, GATHER_CHUNK)
    st, rdp = _sc_stage(V_mem, idx_mat, st_src, pos_mat)

    out_perm = _combine(jlo.reshape(N_CMB), jhi.reshape(N_CMB), st, rdp)
    return _sc_gather(out_perm, pos_mat)


# speedup vs baseline: 1.1262x; 1.0116x over previous
import functools

import jax
import jax.numpy as jnp
from jax import lax
from jax.experimental import pallas as pl
from jax.experimental.pallas import tpu as pltpu
from jax.experimental.pallas import tpu_sc as plsc

N_TOK = 16384
D = 128
N_BITS = 17

BUCKET_SHIFT = 10
N_BUCKETS = 1 << (N_BITS - BUCKET_SHIFT)

SC_CORES = 2
SC_SUBCORES = 16
SC_WORKERS = SC_CORES * SC_SUBCORES
ROWS_PER_WORKER = N_TOK // SC_WORKERS
GATHER_CHUNK = 128
N_CHUNKS = ROWS_PER_WORKER // GATHER_CHUNK

DENSE_BLK = 1024
CMB_BLK = 512
N_CMB = N_TOK // CMB_BLK
ST_W = 256


def _dense_body(x_ref, wkT_ref, bk_ref, wvT_ref, bv_ref, wcT_ref, bc_ref,
                w1T_ref, b1_ref, w2_ref, b2_ref, p_ref,
                vals_ref, g_ref, idx_ref, cnt_ref):
    xb = x_ref[...]
    keys = jnp.dot(xb, wkT_ref[...],
                   preferred_element_type=jnp.float32) + bk_ref[...]
    vals = jnp.dot(xb, wvT_ref[...],
                   preferred_element_type=jnp.float32) + bv_ref[...]
    ctx = jnp.dot(xb, wcT_ref[...],
                  preferred_element_type=jnp.float32) + bc_ref[...]
    h = jax.nn.relu(jnp.dot(ctx, w1T_ref[...],
                            preferred_element_type=jnp.float32) + b1_ref[...])
    glogit = jnp.sum(h * w2_ref[...], axis=1, keepdims=True) + b2_ref[...]
    g = jax.nn.sigmoid(glogit)
    s = jnp.dot(keys, p_ref[...],
                preferred_element_type=jnp.float32)
    bits = (s > 0.0).astype(jnp.int32)
    powers = jnp.left_shift(
        1, lax.broadcasted_iota(jnp.int32, (1, N_BITS), 1))
    idx = jnp.sum(bits * powers, axis=1, keepdims=True)
    vals_ref[...] = vals
    g_ref[...] = g
    idx_ref[...] = idx
    b = idx >> BUCKET_SHIFT
    iota_b = lax.broadcasted_iota(jnp.int32, (1, N_BUCKETS), 1)
    oh = jnp.where(b == iota_b, 1.0, 0.0)
    c0 = jnp.sum(oh[:CMB_BLK], axis=0, keepdims=True)
    c1 = jnp.sum(oh[CMB_BLK:], axis=0, keepdims=True)
    cnt_ref[...] = jnp.concatenate([c0, c1], axis=0).reshape(1, 2, N_BUCKETS)


def _dense_proj(x, WkT, bk, WvT, bv, WcT, bc, W1T, b1, W2r, b2, P):
    n_blk = N_TOK // DENSE_BLK
    full = lambda shape: pl.BlockSpec(shape, lambda i: (0, 0))
    return pl.pallas_call(
        _dense_body,
        grid=(n_blk,),
        in_specs=[
            pl.BlockSpec((DENSE_BLK, D), lambda i: (i, 0)),
            full(WkT.shape), full(bk.shape),
            full(WvT.shape), full(bv.shape),
            full(WcT.shape), full(bc.shape),
            full(W1T.shape), full(b1.shape),
            full(W2r.shape), full(b2.shape),
            full(P.shape),
        ],
        out_specs=[
            pl.BlockSpec((DENSE_BLK, D), lambda i: (i, 0)),
            pl.BlockSpec((DENSE_BLK, 1), lambda i: (i, 0)),
            pl.BlockSpec((DENSE_BLK, 1), lambda i: (i, 0)),
            pl.BlockSpec((1, 2, N_BUCKETS), lambda i: (i, 0, 0)),
        ],
        out_shape=[
            jax.ShapeDtypeStruct((N_TOK, D), jnp.float32),
            jax.ShapeDtypeStruct((N_TOK, 1), jnp.float32),
            jax.ShapeDtypeStruct((N_TOK, 1), jnp.int32),
            jax.ShapeDtypeStruct((n_blk, 2, N_BUCKETS), jnp.float32),
        ],
    )(x, WkT, bk, WvT, bv, WcT, bc, W1T, b1, W2r, b2, P)


def _sc_gather_kernel(table_hbm, idx_hbm, out_hbm, idx_v, rows_v, sem):
    wid = lax.axis_index("s") * SC_CORES + lax.axis_index("c")
    base = wid * ROWS_PER_WORKER
    pltpu.sync_copy(idx_hbm.at[pl.ds(wid * N_CHUNKS, N_CHUNKS)], idx_v)
    for c in range(N_CHUNKS):
        pltpu.async_copy(
            table_hbm.at[idx_v.at[c]],
            rows_v.at[pl.ds(c * GATHER_CHUNK, GATHER_CHUNK)],
            sem,
        ).wait()
    pltpu.sync_copy(rows_v, out_hbm.at[pl.ds(base, ROWS_PER_WORKER)])


def _sc_gather(table, idx_mat):
    mesh = plsc.VectorSubcoreMesh(core_axis_name="c", subcore_axis_name="s")
    kern = functools.partial(
        pl.kernel,
        mesh=mesh,
        out_type=jax.ShapeDtypeStruct((N_TOK, D), jnp.float32),
        scratch_types=[
            pltpu.VMEM((N_CHUNKS, GATHER_CHUNK), jnp.int32),
            pltpu.VMEM((ROWS_PER_WORKER, D), jnp.float32),
            pltpu.SemaphoreType.DMA,
        ],
    )(_sc_gather_kernel)
    return kern(table, idx_mat)


def _sc_stage_kernel(table_hbm, idx_hbm, src_hbm, pos_hbm,
                     st_hbm, rdp_hbm, idx_v, pos_v, rows_v, st_v,
                     sem_a, sem_b):
    wid = lax.axis_index("s") * SC_CORES + lax.axis_index("c")
    base = wid * ROWS_PER_WORKER
    pltpu.sync_copy(idx_hbm.at[pl.ds(wid * N_CHUNKS, N_CHUNKS)], idx_v)
    pltpu.sync_copy(pos_hbm.at[pl.ds(wid * N_CHUNKS, N_CHUNKS)], pos_v)
    for h in range(2):
        ga = [pltpu.async_copy(
                  table_hbm.at[idx_v.at[h * 2 + c]],
                  rows_v.at[pl.ds(c * GATHER_CHUNK, GATHER_CHUNK)],
                  sem_a)
              for c in range(2)]
        ld = pltpu.async_copy(
            src_hbm.at[pl.ds(base + h * 2 * GATHER_CHUNK, 2 * GATHER_CHUNK)],
            st_v, sem_b)
        for c in range(2):
            ga[c].wait()
        sa = [pltpu.async_copy(
                  rows_v.at[pl.ds(c * GATHER_CHUNK, GATHER_CHUNK)],
                  rdp_hbm.at[pos_v.at[h * 2 + c]],
                  sem_a)
              for c in range(2)]
        ld.wait()
        sb = [pltpu.async_copy(
                  st_v.at[pl.ds(c * GATHER_CHUNK, GATHER_CHUNK)],
                  st_hbm.at[pos_v.at[h * 2 + c]],
                  sem_b)
              for c in range(2)]
        for c in range(2):
            sa[c].wait()
            sb[c].wait()


def _sc_stage(V_mem, idx_mat, src, pos_mat):
    mesh = plsc.VectorSubcoreMesh(core_axis_name="c", subcore_axis_name="s")
    kern = functools.partial(
        pl.kernel,
        mesh=mesh,
        out_type=[
            jax.ShapeDtypeStruct((N_TOK, ST_W), jnp.float32),
            jax.ShapeDtypeStruct((N_TOK, D), jnp.float32),
        ],
        scratch_types=[
            pltpu.VMEM((N_CHUNKS, GATHER_CHUNK), jnp.int32),
            pltpu.VMEM((N_CHUNKS, GATHER_CHUNK), jnp.int32),
            pltpu.VMEM((2 * GATHER_CHUNK, D), jnp.float32),
            pltpu.VMEM((2 * GATHER_CHUNK, ST_W), jnp.float32),
            pltpu.SemaphoreType.DMA,
            pltpu.SemaphoreType.DMA,
        ],
    )(_sc_stage_kernel)
    return kern(V_mem, idx_mat, src, pos_mat)


def _p2_body(cnt_ref, base_ref, jlo_ref, jhi_ref):
    def row(b, acc):
        base_ref[pl.ds(b, 1), :, :] = acc.reshape(1, 1, N_BUCKETS)
        return acc + cnt_ref[pl.ds(b, 1), :]

    colsum = lax.fori_loop(0, N_CMB, row,
                           jnp.zeros((1, N_BUCKETS), jnp.float32))
    iota_b = lax.broadcasted_iota(jnp.int32, (1, N_BUCKETS), 1)
    incl = colsum
    sh = 1
    while sh < N_BUCKETS:
        rolled = pltpu.roll(incl, sh, 1)
        incl = incl + jnp.where(iota_b >= sh, rolled, 0.0)
        sh *= 2
    off = jnp.where(iota_b >= 1, pltpu.roll(incl, 1, 1), 0.0)
    base_ref[...] += off.reshape(1, 1, N_BUCKETS)
    rowstart = 512.0 * lax.broadcasted_iota(
        jnp.int32, (N_CMB, 1), 0).astype(jnp.float32)
    hit = jnp.logical_and(off < rowstart + 512.0,
                          off + colsum > rowstart)
    kjlo = jnp.floor(off * (1.0 / 512.0))
    kjhi = jnp.floor((off + colsum - 1.0) * (1.0 / 512.0))
    jlo = jnp.min(jnp.where(hit, kjlo, float(N_CMB)), axis=1, keepdims=True)
    jhi = jnp.max(jnp.where(hit, kjhi, -1.0), axis=1, keepdims=True)
    jlo_ref[...] = jlo.astype(jnp.int32)
    jhi_ref[...] = jhi.astype(jnp.int32)


def _p2(cnt):
    return pl.pallas_call(
        _p2_body,
        grid=(1,),
        in_specs=[pl.BlockSpec((N_CMB, N_BUCKETS), lambda i: (0, 0))],
        out_specs=[
            pl.BlockSpec((N_CMB, 1, N_BUCKETS), lambda i: (0, 0, 0)),
            pl.BlockSpec((N_CMB, 1), lambda i: (0, 0)),
            pl.BlockSpec((N_CMB, 1), lambda i: (0, 0)),
        ],
        out_shape=[
            jax.ShapeDtypeStruct((N_CMB, 1, N_BUCKETS), jnp.float32),
            jax.ShapeDtypeStruct((N_CMB, 1), jnp.int32),
            jax.ShapeDtypeStruct((N_CMB, 1), jnp.int32),
        ],
    )(cnt)


def _p3_body(idx_ref, g_ref, vals_ref, base_ref, st_ref, pos_ref):
    idx = idx_ref[...]
    b = idx >> BUCKET_SHIFT
    iota_b = lax.broadcasted_iota(jnp.int32, (1, N_BUCKETS), 1)
    oh_f = jnp.where(b == iota_b, 1.0, 0.0)
    oh_b = oh_f.astype(jnp.bfloat16)
    r_io = lax.broadcasted_iota(jnp.int32, (CMB_BLK, 1), 0)
    c_io = lax.broadcasted_iota(jnp.int32, (1, CMB_BLK), 1)
    ls = jnp.where(r_io > c_io, 1.0, 0.0).astype(jnp.bfloat16)
    rank_mat = jnp.dot(ls, oh_b, preferred_element_type=jnp.float32)
    rank = jnp.sum(rank_mat * oh_f, axis=1, keepdims=True)
    base_row = base_ref[...].reshape(1, N_BUCKETS)
    base_sel = jnp.sum(base_row * oh_f, axis=1, keepdims=True)
    pos_ref[...] = (base_sel + rank).astype(jnp.int32)
    pad = jnp.zeros((CMB_BLK, ST_W - D - 2), jnp.float32)
    st_ref[...] = jnp.concatenate(
        [vals_ref[...], idx.astype(jnp.float32), g_ref[...], pad], axis=1)


def _p3(idx, g, vals, base):
    return pl.pallas_call(
        _p3_body,
        grid=(N_CMB,),
        in_specs=[
            pl.BlockSpec((CMB_BLK, 1), lambda i: (i, 0)),
            pl.BlockSpec((CMB_BLK, 1), lambda i: (i, 0)),
            pl.BlockSpec((CMB_BLK, D), lambda i: (i, 0)),
            pl.BlockSpec((1, 1, N_BUCKETS), lambda i: (i, 0, 0)),
        ],
        out_specs=[
            pl.BlockSpec((CMB_BLK, ST_W), lambda i: (i, 0)),
            pl.BlockSpec((CMB_BLK, 1), lambda i: (i, 0)),
        ],
        out_shape=[
            jax.ShapeDtypeStruct((N_TOK, ST_W), jnp.float32),
            jax.ShapeDtypeStruct((N_TOK, 1), jnp.int32),
        ],
    )(idx, g, vals, base)


def _bits_of(idx_f):
    ii = idx_f.astype(jnp.int32)
    iota = lax.broadcasted_iota(jnp.int32, (1, N_BITS), 1)
    bits = jnp.where(jnp.bitwise_and(ii >> iota, 1) == 1, 1.0, 0.0)
    return bits, jnp.sum(bits, axis=1, keepdims=True)


def _combine_body(jlo_ref, jhi_ref, st_ref, rdp_ref, out_ref, acc_ref):
    i = pl.program_id(0)
    sti = st_ref[pl.ds(i * CMB_BLK, CMB_BLK), :]
    bits_i, s_i = _bits_of(sti[:, D:D + 1])
    ua_i = jnp.concatenate(
        [2.0 * bits_i, jnp.ones((CMB_BLK, 1), jnp.float32)],
        axis=1).astype(jnp.bfloat16)
    acc_ref[...] = jnp.zeros((CMB_BLK, 2 * D), jnp.float32)

    def jbody(j, _):
        stj = st_ref[pl.ds(j * CMB_BLK, CMB_BLK), :]
        bits_j, s_j = _bits_of(stj[:, D:D + 1])
        ub_j = jnp.concatenate([bits_j, -s_j], axis=1).astype(jnp.bfloat16)
        m = lax.dot_general(ua_i, ub_j, (((1,), (1,)), ((), ())),
                            preferred_element_type=jnp.float32)
        mask = jnp.maximum(m - s_i + 1.0, 0.0).astype(jnp.bfloat16)
        dj = stj[:, D + 1:D + 2] * (
            stj[:, :D] - rdp_ref[pl.ds(j * CMB_BLK, CMB_BLK), :])
        hi = dj.astype(jnp.bfloat16)
        lo = (dj - hi.astype(jnp.float32)).astype(jnp.bfloat16)
        d2 = jnp.concatenate([hi, lo], axis=1)
        acc_ref[...] += jnp.dot(mask, d2, preferred_element_type=jnp.float32)
        return 0

    lax.fori_loop(jlo_ref[i], jhi_ref[i] + 1, jbody, 0)
    acc = acc_ref[...]
    out_ref[...] = (rdp_ref[pl.ds(i * CMB_BLK, CMB_BLK), :]
                    + acc[:, :D] + acc[:, D:])


def _combine(jlo, jhi, st, rdp):
    grid_spec = pltpu.PrefetchScalarGridSpec(
        num_scalar_prefetch=2,
        grid=(N_CMB,),
        in_specs=[
            pl.BlockSpec((N_TOK, ST_W), lambda i, jlo_r, jhi_r: (0, 0)),
            pl.BlockSpec((N_TOK, D), lambda i, jlo_r, jhi_r: (0, 0)),
        ],
        out_specs=pl.BlockSpec((CMB_BLK, D), lambda i, jlo_r, jhi_r: (i, 0)),
        scratch_shapes=[pltpu.VMEM((CMB_BLK, 2 * D), jnp.float32)],
    )
    return pl.pallas_call(
        _combine_body,
        grid_spec=grid_spec,
        out_shape=jax.ShapeDtypeStruct((N_TOK, D), jnp.float32),
    )(jlo, jhi, st, rdp)


def kernel(x, Wk, bk, Wv, bv, Wc, bc, W1, b1, W2, b2, P, V_mem):
    WkT = Wk.T
    WvT = Wv.T
    WcT = Wc.T
    W1T = W1.T
    W2r = W2.reshape(1, -1)
    bk2 = bk.reshape(1, -1)
    bv2 = bv.reshape(1, -1)
    bc2 = bc.reshape(1, -1)
    b12 = b1.reshape(1, -1)
    b22 = b2.reshape(1, 1)

    vals, g, idx, cnt3 = _dense_proj(x, WkT, bk2, WvT, bv2, WcT, bc2,
                                     W1T, b12, W2r, b22, P)
    idx_mat = idx.reshape(SC_WORKERS * N_CHUNKS, GATHER_CHUNK)

    cnt = cnt3.reshape(N_CMB, N_BUCKETS)
    base, jlo, jhi = _p2(cnt)
    st_src, pos = _p3(idx, g, vals, base)
    pos_mat = pos.reshape(SC_WORKERS * N_CHUNKS, GATHER_CHUNK)
    st, rdp = _sc_stage(V_mem, idx_mat, st_src, pos_mat)

    out_perm = _combine(jlo.reshape(N_CMB), jhi.reshape(N_CMB), st, rdp)
    return _sc_gather(out_perm, pos_mat)


# speedup vs baseline: 1.1368x; 1.0095x over previous
import functools

import jax
import jax.numpy as jnp
from jax import lax
from jax.experimental import pallas as pl
from jax.experimental.pallas import tpu as pltpu
from jax.experimental.pallas import tpu_sc as plsc

N_TOK = 16384
D = 128
N_BITS = 17

BUCKET_SHIFT = 10
N_BUCKETS = 1 << (N_BITS - BUCKET_SHIFT)

SC_CORES = 2
SC_SUBCORES = 16
SC_WORKERS = SC_CORES * SC_SUBCORES
ROWS_PER_WORKER = N_TOK // SC_WORKERS
GATHER_CHUNK = 128
N_CHUNKS = ROWS_PER_WORKER // GATHER_CHUNK

DENSE_BLK = 1024
CMB_BLK = 512
N_CMB = N_TOK // CMB_BLK
ST_W = 256


def _dense_body(x_ref, wkT_ref, bk_ref, wvT_ref, bv_ref, wcT_ref, bc_ref,
                w1T_ref, b1_ref, w2_ref, b2_ref, p_ref,
                vals_ref, g_ref, idx_ref, cnt_ref):
    xb = x_ref[...]
    keys = jnp.dot(xb, wkT_ref[...],
                   preferred_element_type=jnp.float32) + bk_ref[...]
    vals = jnp.dot(xb, wvT_ref[...],
                   preferred_element_type=jnp.float32) + bv_ref[...]
    ctx = jnp.dot(xb, wcT_ref[...],
                  preferred_element_type=jnp.float32) + bc_ref[...]
    h = jax.nn.relu(jnp.dot(ctx, w1T_ref[...],
                            preferred_element_type=jnp.float32) + b1_ref[...])
    glogit = jnp.sum(h * w2_ref[...], axis=1, keepdims=True) + b2_ref[...]
    g = jax.nn.sigmoid(glogit)
    s = jnp.dot(keys, p_ref[...],
                preferred_element_type=jnp.float32)
    bits = (s > 0.0).astype(jnp.int32)
    powers = jnp.left_shift(
        1, lax.broadcasted_iota(jnp.int32, (1, N_BITS), 1))
    idx = jnp.sum(bits * powers, axis=1, keepdims=True)
    vals_ref[...] = vals
    g_ref[...] = g
    idx_ref[...] = idx
    b = idx >> BUCKET_SHIFT
    iota_b = lax.broadcasted_iota(jnp.int32, (1, N_BUCKETS), 1)
    oh = jnp.where(b == iota_b, 1.0, 0.0)
    c0 = jnp.sum(oh[:CMB_BLK], axis=0, keepdims=True)
    c1 = jnp.sum(oh[CMB_BLK:], axis=0, keepdims=True)
    cnt_ref[...] = jnp.concatenate([c0, c1], axis=0).reshape(1, 2, N_BUCKETS)


def _dense_proj(x, WkT, bk, WvT, bv, WcT, bc, W1T, b1, W2r, b2, P):
    n_blk = N_TOK // DENSE_BLK
    full = lambda shape: pl.BlockSpec(shape, lambda i: (0, 0))
    return pl.pallas_call(
        _dense_body,
        grid=(n_blk,),
        in_specs=[
            pl.BlockSpec((DENSE_BLK, D), lambda i: (i, 0)),
            full(WkT.shape), full(bk.shape),
            full(WvT.shape), full(bv.shape),
            full(WcT.shape), full(bc.shape),
            full(W1T.shape), full(b1.shape),
            full(W2r.shape), full(b2.shape),
            full(P.shape),
        ],
        out_specs=[
            pl.BlockSpec((DENSE_BLK, D), lambda i: (i, 0)),
            pl.BlockSpec((DENSE_BLK, 1), lambda i: (i, 0)),
            pl.BlockSpec((DENSE_BLK, 1), lambda i: (i, 0)),
            pl.BlockSpec((1, 2, N_BUCKETS), lambda i: (i, 0, 0)),
        ],
        out_shape=[
            jax.ShapeDtypeStruct((N_TOK, D), jnp.float32),
            jax.ShapeDtypeStruct((N_TOK, 1), jnp.float32),
            jax.ShapeDtypeStruct((N_TOK, 1), jnp.int32),
            jax.ShapeDtypeStruct((n_blk, 2, N_BUCKETS), jnp.float32),
        ],
    )(x, WkT, bk, WvT, bv, WcT, bc, W1T, b1, W2r, b2, P)


def _sc_gather_kernel(table_hbm, idx_hbm, out_hbm, idx_v, rows_v, sem):
    wid = lax.axis_index("s") * SC_CORES + lax.axis_index("c")
    base = wid * ROWS_PER_WORKER
    pltpu.sync_copy(idx_hbm.at[pl.ds(wid * N_CHUNKS, N_CHUNKS)], idx_v)
    cps = [pltpu.async_copy(
               table_hbm.at[idx_v.at[c]],
               rows_v.at[pl.ds(c * GATHER_CHUNK, GATHER_CHUNK)],
               sem)
           for c in range(N_CHUNKS)]
    for cp in cps:
        cp.wait()
    pltpu.sync_copy(rows_v, out_hbm.at[pl.ds(base, ROWS_PER_WORKER)])


def _sc_gather(table, idx_mat):
    mesh = plsc.VectorSubcoreMesh(core_axis_name="c", subcore_axis_name="s")
    kern = functools.partial(
        pl.kernel,
        mesh=mesh,
        out_type=jax.ShapeDtypeStruct((N_TOK, D), jnp.float32),
        scratch_types=[
            pltpu.VMEM((N_CHUNKS, GATHER_CHUNK), jnp.int32),
            pltpu.VMEM((ROWS_PER_WORKER, D), jnp.float32),
            pltpu.SemaphoreType.DMA,
        ],
    )(_sc_gather_kernel)
    return kern(table, idx_mat)


def _sc_stage_kernel(table_hbm, idx_hbm, src_hbm, pos_hbm,
                     st_hbm, rdp_hbm, idx_v, pos_v, rows_v, st_v,
                     sem_a, sem_b):
    wid = lax.axis_index("s") * SC_CORES + lax.axis_index("c")
    base = wid * ROWS_PER_WORKER
    pltpu.sync_copy(idx_hbm.at[pl.ds(wid * N_CHUNKS, N_CHUNKS)], idx_v)
    pltpu.sync_copy(pos_hbm.at[pl.ds(wid * N_CHUNKS, N_CHUNKS)], pos_v)
    for h in range(2):
        ga = [pltpu.async_copy(
                  table_hbm.at[idx_v.at[h * 2 + c]],
                  rows_v.at[pl.ds(c * GATHER_CHUNK, GATHER_CHUNK)],
                  sem_a)
              for c in range(2)]
        ld = pltpu.async_copy(
            src_hbm.at[pl.ds(base + h * 2 * GATHER_CHUNK, 2 * GATHER_CHUNK)],
            st_v, sem_b)
        for c in range(2):
            ga[c].wait()
        sa = [pltpu.async_copy(
                  rows_v.at[pl.ds(c * GATHER_CHUNK, GATHER_CHUNK)],
                  rdp_hbm.at[pos_v.at[h * 2 + c]],
                  sem_a)
              for c in range(2)]
        ld.wait()
        sb = [pltpu.async_copy(
                  st_v.at[pl.ds(c * GATHER_CHUNK, GATHER_CHUNK)],
                  st_hbm.at[pos_v.at[h * 2 + c]],
                  sem_b)
              for c in range(2)]
        for c in range(2):
            sa[c].wait()
            sb[c].wait()


def _sc_stage(V_mem, idx_mat, src, pos_mat):
    mesh = plsc.VectorSubcoreMesh(core_axis_name="c", subcore_axis_name="s")
    kern = functools.partial(
        pl.kernel,
        mesh=mesh,
        out_type=[
            jax.ShapeDtypeStruct((N_TOK, ST_W), jnp.float32),
            jax.ShapeDtypeStruct((N_TOK, D), jnp.float32),
        ],
        scratch_types=[
            pltpu.VMEM((N_CHUNKS, GATHER_CHUNK), jnp.int32),
            pltpu.VMEM((N_CHUNKS, GATHER_CHUNK), jnp.int32),
            pltpu.VMEM((2 * GATHER_CHUNK, D), jnp.float32),
            pltpu.VMEM((2 * GATHER_CHUNK, ST_W), jnp.float32),
            pltpu.SemaphoreType.DMA,
            pltpu.SemaphoreType.DMA,
        ],
    )(_sc_stage_kernel)
    return kern(V_mem, idx_mat, src, pos_mat)


def _p2_body(cnt_ref, base_ref, jlo_ref, jhi_ref):
    def row(b, acc):
        base_ref[pl.ds(b, 1), :, :] = acc.reshape(1, 1, N_BUCKETS)
        return acc + cnt_ref[pl.ds(b, 1), :]

    colsum = lax.fori_loop(0, N_CMB, row,
                           jnp.zeros((1, N_BUCKETS), jnp.float32))
    iota_b = lax.broadcasted_iota(jnp.int32, (1, N_BUCKETS), 1)
    incl = colsum
    sh = 1
    while sh < N_BUCKETS:
        rolled = pltpu.roll(incl, sh, 1)
        incl = incl + jnp.where(iota_b >= sh, rolled, 0.0)
        sh *= 2
    off = jnp.where(iota_b >= 1, pltpu.roll(incl, 1, 1), 0.0)
    base_ref[...] += off.reshape(1, 1, N_BUCKETS)
    rowstart = 512.0 * lax.broadcasted_iota(
        jnp.int32, (N_CMB, 1), 0).astype(jnp.float32)
    hit = jnp.logical_and(off < rowstart + 512.0,
                          off + colsum > rowstart)
    kjlo = jnp.floor(off * (1.0 / 512.0))
    kjhi = jnp.floor((off + colsum - 1.0) * (1.0 / 512.0))
    jlo = jnp.min(jnp.where(hit, kjlo, float(N_CMB)), axis=1, keepdims=True)
    jhi = jnp.max(jnp.where(hit, kjhi, -1.0), axis=1, keepdims=True)
    jlo_ref[...] = jlo.astype(jnp.int32)
    jhi_ref[...] = jhi.astype(jnp.int32)


def _p2(cnt):
    return pl.pallas_call(
        _p2_body,
        grid=(1,),
        in_specs=[pl.BlockSpec((N_CMB, N_BUCKETS), lambda i: (0, 0))],
        out_specs=[
            pl.BlockSpec((N_CMB, 1, N_BUCKETS), lambda i: (0, 0, 0)),
            pl.BlockSpec((N_CMB, 1), lambda i: (0, 0)),
            pl.BlockSpec((N_CMB, 1), lambda i: (0, 0)),
        ],
        out_shape=[
            jax.ShapeDtypeStruct((N_CMB, 1, N_BUCKETS), jnp.float32),
            jax.ShapeDtypeStruct((N_CMB, 1), jnp.int32),
            jax.ShapeDtypeStruct((N_CMB, 1), jnp.int32),
        ],
    )(cnt)


def _p3_body(idx_ref, g_ref, vals_ref, base_ref, st_ref, pos_ref):
    idx = idx_ref[...]
    b = idx >> BUCKET_SHIFT
    iota_b = lax.broadcasted_iota(jnp.int32, (1, N_BUCKETS), 1)
    oh_f = jnp.where(b == iota_b, 1.0, 0.0)
    oh_b = oh_f.astype(jnp.bfloat16)
    r_io = lax.broadcasted_iota(jnp.int32, (CMB_BLK, 1), 0)
    c_io = lax.broadcasted_iota(jnp.int32, (1, CMB_BLK), 1)
    ls = jnp.where(r_io > c_io, 1.0, 0.0).astype(jnp.bfloat16)
    rank_mat = jnp.dot(ls, oh_b, preferred_element_type=jnp.float32)
    rank = jnp.sum(rank_mat * oh_f, axis=1, keepdims=True)
    base_row = base_ref[...].reshape(1, N_BUCKETS)
    base_sel = jnp.sum(base_row * oh_f, axis=1, keepdims=True)
    pos_ref[...] = (base_sel + rank).astype(jnp.int32)
    pad = jnp.zeros((CMB_BLK, ST_W - D - 2), jnp.float32)
    st_ref[...] = jnp.concatenate(
        [vals_ref[...], idx.astype(jnp.float32), g_ref[...], pad], axis=1)


def _p3(idx, g, vals, base):
    return pl.pallas_call(
        _p3_body,
        grid=(N_CMB,),
        in_specs=[
            pl.BlockSpec((CMB_BLK, 1), lambda i: (i, 0)),
            pl.BlockSpec((CMB_BLK, 1), lambda i: (i, 0)),
            pl.BlockSpec((CMB_BLK, D), lambda i: (i, 0)),
            pl.BlockSpec((1, 1, N_BUCKETS), lambda i: (i, 0, 0)),
        ],
        out_specs=[
            pl.BlockSpec((CMB_BLK, ST_W), lambda i: (i, 0)),
            pl.BlockSpec((CMB_BLK, 1), lambda i: (i, 0)),
        ],
        out_shape=[
            jax.ShapeDtypeStruct((N_TOK, ST_W), jnp.float32),
            jax.ShapeDtypeStruct((N_TOK, 1), jnp.int32),
        ],
    )(idx, g, vals, base)


def _bits_of(idx_f):
    ii = idx_f.astype(jnp.int32)
    iota = lax.broadcasted_iota(jnp.int32, (1, N_BITS), 1)
    bits = jnp.where(jnp.bitwise_and(ii >> iota, 1) == 1, 1.0, 0.0)
    return bits, jnp.sum(bits, axis=1, keepdims=True)


def _combine_body(jlo_ref, jhi_ref, st_ref, rdp_ref, out_ref, acc_ref):
    i = pl.program_id(0)
    sti = st_ref[pl.ds(i * CMB_BLK, CMB_BLK), :]
    bits_i, s_i = _bits_of(sti[:, D:D + 1])
    ua_i = jnp.concatenate(
        [2.0 * bits_i, jnp.ones((CMB_BLK, 1), jnp.float32)],
        axis=1).astype(jnp.bfloat16)
    acc_ref[...] = jnp.zeros((CMB_BLK, 2 * D), jnp.float32)

    def jbody(j, _):
        stj = st_ref[pl.ds(j * CMB_BLK, CMB_BLK), :]
        bits_j, s_j = _bits_of(stj[:, D:D + 1])
        ub_j = jnp.concatenate([bits_j, -s_j], axis=1).astype(jnp.bfloat16)
        m = lax.dot_general(ua_i, ub_j, (((1,), (1,)), ((), ())),
                            preferred_element_type=jnp.float32)
        mask = jnp.maximum(m - s_i + 1.0, 0.0).astype(jnp.bfloat16)
        dj = stj[:, D + 1:D + 2] * (
            stj[:, :D] - rdp_ref[pl.ds(j * CMB_BLK, CMB_BLK), :])
        hi = dj.astype(jnp.bfloat16)
        lo = (dj - hi.astype(jnp.float32)).astype(jnp.bfloat16)
        d2 = jnp.concatenate([hi, lo], axis=1)
        acc_ref[...] += jnp.dot(mask, d2, preferred_element_type=jnp.float32)
        return 0

    lax.fori_loop(jlo_ref[i], jhi_ref[i] + 1, jbody, 0)
    acc = acc_ref[...]
    out_ref[...] = (rdp_ref[pl.ds(i * CMB_BLK, CMB_BLK), :]
                    + acc[:, :D] + acc[:, D:])


def _combine(jlo, jhi, st, rdp):
    grid_spec = pltpu.PrefetchScalarGridSpec(
        num_scalar_prefetch=2,
        grid=(N_CMB,),
        in_specs=[
            pl.BlockSpec((N_TOK, ST_W), lambda i, jlo_r, jhi_r: (0, 0)),
            pl.BlockSpec((N_TOK, D), lambda i, jlo_r, jhi_r: (0, 0)),
        ],
        out_specs=pl.BlockSpec((CMB_BLK, D), lambda i, jlo_r, jhi_r: (i, 0)),
        scratch_shapes=[pltpu.VMEM((CMB_BLK, 2 * D), jnp.float32)],
    )
    return pl.pallas_call(
        _combine_body,
        grid_spec=grid_spec,
        out_shape=jax.ShapeDtypeStruct((N_TOK, D), jnp.float32),
    )(jlo, jhi, st, rdp)


def kernel(x, Wk, bk, Wv, bv, Wc, bc, W1, b1, W2, b2, P, V_mem):
    WkT = Wk.T
    WvT = Wv.T
    WcT = Wc.T
    W1T = W1.T
    W2r = W2.reshape(1, -1)
    bk2 = bk.reshape(1, -1)
    bv2 = bv.reshape(1, -1)
    bc2 = bc.reshape(1, -1)
    b12 = b1.reshape(1, -1)
    b22 = b2.reshape(1, 1)

    vals, g, idx, cnt3 = _dense_proj(x, WkT, bk2, WvT, bv2, WcT, bc2,
                                     W1T, b12, W2r, b22, P)
    idx_mat = idx.reshape(SC_WORKERS * N_CHUNKS, GATHER_CHUNK)

    cnt = cnt3.reshape(N_CMB, N_BUCKETS)
    base, jlo, jhi = _p2(cnt)
    st_src, pos = _p3(idx, g, vals, base)
    pos_mat = pos.reshape(SC_WORKERS * N_CHUNKS, GATHER_CHUNK)
    st, rdp = _sc_stage(V_mem, idx_mat, st_src, pos_mat)

    out_perm = _combine(jlo.reshape(N_CMB), jhi.reshape(N_CMB), st, rdp)
    return _sc_gather(out_perm, pos_mat)
